# speedup vs baseline: 1.6554x; 1.6554x over previous
import functools

import jax
import jax.numpy as jnp
from jax import lax
from jax.experimental import pallas as pl
from jax.experimental.pallas import tpu as pltpu
from jax.experimental.pallas import tpu_sc as plsc

PRE = 128
D = 18432
B = 2048
NW = 32
PER_W = B // NW
C = 2
NCH = PER_W // C

_mesh = plsc.VectorSubcoreMesh(core_axis_name="c", subcore_axis_name="s")


@functools.partial(
    pl.kernel,
    mesh=_mesh,
    out_type=jax.ShapeDtypeStruct((B, D), jnp.float32),
    scratch_types=[
        pltpu.VMEM((NCH, C), jnp.int32),
        pltpu.VMEM((C, D), jnp.float32),
        pltpu.VMEM((C, D), jnp.float32),
        pltpu.SemaphoreType.DMA,
        pltpu.SemaphoreType.DMA,
        pltpu.SemaphoreType.DMA,
        pltpu.SemaphoreType.DMA,
    ],
)
def _gather_kernel(idx_hbm, table_hbm, out_hbm, idx_v, buf0, buf1, g0, g1, p0, p1):
    wid = lax.axis_index("s") * 2 + lax.axis_index("c")
    base = wid * PER_W
    pltpu.sync_copy(idx_hbm.at[pl.ds(wid * NCH, NCH)], idx_v)

    bufs = (buf0, buf1)
    gsems = (g0, g1)
    psems = (p0, p1)

    pltpu.async_copy(table_hbm.at[idx_v.at[0]], buf0, g0)
    pltpu.async_copy(table_hbm.at[idx_v.at[1]], buf1, g1)

    def body(i, carry):
        for b in range(2):
            k = i * 2 + b
            buf, gs, ps = bufs[b], gsems[b], psems[b]
            pltpu.make_async_copy(table_hbm.at[idx_v.at[0]], buf, gs).wait()
            pltpu.async_copy(buf, out_hbm.at[pl.ds(base + k * C, C)], ps)

            @pl.when(k + 2 < NCH)
            def _():
                pltpu.make_async_copy(buf, out_hbm.at[pl.ds(base, C)], ps).wait()
                pltpu.async_copy(table_hbm.at[idx_v.at[k + 2]], buf, gs)

        return carry

    lax.fori_loop(0, NCH // 2, body, 0)

    pltpu.make_async_copy(buf0, out_hbm.at[pl.ds(base, C)], p0).wait()
    pltpu.make_async_copy(buf1, out_hbm.at[pl.ds(base, C)], p1).wait()


def kernel(prefix, embedding):
    idx = prefix.reshape(NW * NCH, C)
    out = _gather_kernel(idx, embedding)
    return out.reshape(prefix.shape[0], prefix.shape[1], D)
